# baseline (speedup 1.0000x reference)
import functools

import jax
import jax.numpy as jnp
from jax import lax
from jax.experimental import pallas as pl
from jax.experimental.pallas import tpu as pltpu
from jax.experimental.pallas import tpu_sc as plsc

NUM_SITES = 1000000
EMBEDDING_DIM = 64
BATCH = 16384

_info = plsc.get_sparse_core_info()
_NC, _NS = _info.num_cores, _info.num_subcores
_NW = _NC * _NS
_LANES = 16

_NT = (NUM_SITES + 127) // 128
_NBW = (_NT + _NW - 1) // _NW
_QUAD = 4
_BLK_W = _QUAD * 128

_RING = 16
_CAP = BATCH + _LANES
_CNT_CAP = _NBW + 2 * _LANES


def _iota16():
    return lax.iota(jnp.int32, _LANES)


def _full16(x):
    return jnp.full((_LANES,), x, jnp.int32)


def _make_kernel():
    mesh = plsc.VectorSubcoreMesh(core_axis_name="c", subcore_axis_name="s")

    @functools.partial(
        pl.kernel,
        mesh=mesh,
        out_type=jax.ShapeDtypeStruct((BATCH, EMBEDDING_DIM), jnp.float32),
        scratch_types=[
            pltpu.VMEM((BATCH,), jnp.int32),
            pltpu.VMEM((_CAP + _LANES,), jnp.int32),
            pltpu.VMEM((_CAP + _LANES,), jnp.int32),
            pltpu.VMEM((_CNT_CAP,), jnp.int32),
            pltpu.VMEM((_CNT_CAP,), jnp.int32),
            pltpu.VMEM((2, EMBEDDING_DIM, _BLK_W), jnp.float32),
            pltpu.VMEM((_RING, EMBEDDING_DIM), jnp.float32),
            pltpu.SemaphoreType.DMA,
            pltpu.SemaphoreType.DMA,
            pltpu.SemaphoreType.DMA,
        ],
        compiler_params=pltpu.CompilerParams(needs_layout_passes=False),
    )
    def gather_kernel(
        idx_hbm,
        table_hbm,
        out_hbm,
        idx_all,
        my_pk,
        srt_pk,
        counts,
        cursors,
        blk,
        ring,
        sem_row,
        sem_b0,
        sem_b1,
    ):
        wid = lax.axis_index("s") * _NC + lax.axis_index("c")
        lo = wid * _NBW
        nblk = jnp.minimum(_NBW, _NT - lo)
        hi = lo + nblk
        rel0 = lo * 128
        iota = _iota16()
        lane0 = iota == 0
        ones16 = _full16(1)
        dump_v = _CAP + iota
        dump_c = (_NBW + _LANES) + iota

        max_base = _NT * 128 - _BLK_W

        def quad_base(p):
            return jnp.minimum((lo + _QUAD * p) * 128, max_base)

        def fetch_quad(p, buf, sem):
            base = pl.multiple_of(quad_base(p), 128)
            pltpu.async_copy(
                table_hbm.at[:, pl.ds(base, _BLK_W)], blk.at[buf], sem
            )

        fetch_quad(0, 0, sem_b0)
        fetch_quad(1, 1, sem_b1)

        pltpu.sync_copy(idx_hbm, idx_all)

        def compact_body(g, count):
            v = idx_all[pl.ds(g * _LANES, _LANES)]
            c = lax.shift_right_logical(v, 7)
            m = (c >= lo) & (c < hi)
            mi = m.astype(jnp.int32)
            pref = plsc.cumsum(mi)
            tgt = jnp.where(m, count + pref - mi, dump_v)
            pk = jnp.bitwise_or(
                lax.shift_left(v - rel0, 14), iota + g * _LANES
            )
            plsc.store_scatter(my_pk, [tgt], pk)
            return count + pref[_LANES - 1]

        n_mine = lax.fori_loop(0, BATCH // _LANES, compact_body, 0)

        for g in range(_CNT_CAP // _LANES):
            counts[pl.ds(g * _LANES, _LANES)] = jnp.zeros((_LANES,), jnp.int32)

        def hist_body(i, carry):
            w = my_pk[pl.ds(i, _LANES)][0]
            c0 = lax.shift_right_logical(w, 21)
            tgt = jnp.where(lane0, _full16(c0), dump_c)
            plsc.addupdate_scatter(counts, [tgt], ones16)
            return carry

        lax.fori_loop(0, n_mine, hist_body, 0)

        carry = jnp.int32(0)
        for g in range(_NBW // _LANES + 1):
            off = g * _LANES
            v = counts[pl.ds(off, _LANES)]
            s = plsc.cumsum(v)
            cursors[pl.ds(off, _LANES)] = s - v + carry
            carry = carry + s[_LANES - 1]

        def place_body(i, carry):
            wv = my_pk[pl.ds(i, _LANES)]
            c0 = lax.shift_right_logical(wv[0], 21)
            c0v = _full16(c0)
            curv = plsc.load_gather(cursors, [c0v])
            tgt = jnp.where(lane0, curv, dump_v)
            plsc.store_scatter(srt_pk, [tgt], _full16(wv[0]))
            tgt2 = jnp.where(lane0, c0v, dump_c)
            plsc.addupdate_scatter(cursors, [tgt2], ones16)
            return carry

        lax.fori_loop(0, n_mine, place_body, 0)

        nquad = lax.div(nblk + _QUAD - 1, _QUAD)

        def drain_one():
            pltpu.make_async_copy(
                ring.at[pl.ds(0, 1), :], out_hbm.at[pl.ds(0, 1), :], sem_row
            ).wait()

        def wait_quad(buf, sem):
            pltpu.make_async_copy(
                table_hbm.at[:, pl.ds(0, _BLK_W)], blk.at[buf], sem
            ).wait()

        def make_site_body(buf, base_rel):
            def site_body(j, carry):
                w = srt_pk[pl.ds(j, _LANES)][0]
                pos = jnp.bitwise_and(w, 16383)
                l = lax.shift_right_logical(w, 14) - base_rel
                slot = jnp.bitwise_and(j, _RING - 1)

                @pl.when(j >= _RING)
                def _():
                    drain_one()

                lv = _full16(l)
                for g in range(EMBEDDING_DIM // _LANES):
                    fv = _iota16() + g * _LANES
                    row = plsc.load_gather(blk.at[buf], [fv, lv])
                    ring[slot, pl.ds(g * _LANES, _LANES)] = row
                pltpu.async_copy(
                    ring.at[pl.ds(slot, 1), :],
                    out_hbm.at[pl.ds(pos, 1), :],
                    sem_row,
                )
                return carry

            return site_body

        def process_quad(p, buf):
            base_rel = quad_base(p) - rel0
            for half in range(_QUAD):
                b = _QUAD * p + half

                @pl.when(b < nblk)
                def _():
                    end = cursors[pl.ds(b, _LANES)][0]
                    cnt = counts[pl.ds(b, _LANES)][0]
                    lax.fori_loop(
                        end - cnt, end, make_site_body(buf, base_rel), 0
                    )

        def quad_loop(q, carry):
            p0 = 2 * q
            p1 = 2 * q + 1

            @pl.when(p0 < nquad)
            def _():
                wait_quad(0, sem_b0)
                process_quad(p0, 0)

            @pl.when(p0 + 2 < nquad)
            def _():
                fetch_quad(p0 + 2, 0, sem_b0)

            @pl.when(p1 < nquad)
            def _():
                wait_quad(1, sem_b1)
                process_quad(p1, 1)

            @pl.when(p1 + 2 < nquad)
            def _():
                fetch_quad(p1 + 2, 1, sem_b1)

            return carry

        lax.fori_loop(0, (nquad + 1) // 2, quad_loop, 0)

        for k in range(_RING):

            @pl.when(n_mine > k)
            def _():
                drain_one()

    return gather_kernel


_gather = jax.jit(_make_kernel())


def kernel(site_ids, embedding_weight):
    return _gather(site_ids.astype(jnp.int32), embedding_weight.T)


# speedup vs baseline: 1.0969x; 1.0969x over previous
import functools

import jax
import jax.numpy as jnp
from jax import lax
from jax.experimental import pallas as pl
from jax.experimental.pallas import tpu as pltpu
from jax.experimental.pallas import tpu_sc as plsc

NUM_SITES = 1000000
EMBEDDING_DIM = 64
BATCH = 16384

_info = plsc.get_sparse_core_info()
_NC, _NS = _info.num_cores, _info.num_subcores
_NW = _NC * _NS
_LANES = 16

_NT = (NUM_SITES + 127) // 128
_NBW = (_NT + _NW - 1) // _NW
_QUAD = 2
_BLK_W = _QUAD * 128
_NBUF = 4

_RING = 16
_CAP = BATCH + _LANES
_CNT_CAP = _NBW + 2 * _LANES


def _iota16():
    return lax.iota(jnp.int32, _LANES)


def _full16(x):
    return jnp.full((_LANES,), x, jnp.int32)


def _make_kernel():
    mesh = plsc.VectorSubcoreMesh(core_axis_name="c", subcore_axis_name="s")

    @functools.partial(
        pl.kernel,
        mesh=mesh,
        out_type=jax.ShapeDtypeStruct((BATCH, EMBEDDING_DIM), jnp.float32),
        scratch_types=[
            pltpu.VMEM((BATCH,), jnp.int32),
            pltpu.VMEM((_CAP + _LANES,), jnp.int32),
            pltpu.VMEM((_CAP + _LANES,), jnp.int32),
            pltpu.VMEM((_CNT_CAP,), jnp.int32),
            pltpu.VMEM((_CNT_CAP,), jnp.int32),
            pltpu.VMEM((_NBUF, EMBEDDING_DIM, _BLK_W), jnp.float32),
            pltpu.VMEM((_RING, EMBEDDING_DIM), jnp.float32),
            pltpu.SemaphoreType.DMA,
            pltpu.SemaphoreType.DMA,
            pltpu.SemaphoreType.DMA,
            pltpu.SemaphoreType.DMA,
            pltpu.SemaphoreType.DMA,
        ],
        compiler_params=pltpu.CompilerParams(needs_layout_passes=False),
    )
    def gather_kernel(
        idx_hbm,
        table_hbm,
        out_hbm,
        idx_all,
        my_pk,
        srt_pk,
        counts,
        cursors,
        blk,
        ring,
        sem_row,
        sem_b0,
        sem_b1,
        sem_b2,
        sem_b3,
    ):
        wid = lax.axis_index("s") * _NC + lax.axis_index("c")
        lo = wid * _NBW
        nblk = jnp.minimum(_NBW, _NT - lo)
        hi = lo + nblk
        rel0 = lo * 128
        iota = _iota16()
        lane0 = iota == 0
        ones16 = _full16(1)
        dump_v = _CAP + iota
        dump_c = (_NBW + _LANES) + iota

        max_base = _NT * 128 - _BLK_W

        def quad_base(p):
            return jnp.minimum((lo + _QUAD * p) * 128, max_base)

        def fetch_quad(p, buf, sem):
            base = pl.multiple_of(quad_base(p), 128)
            pltpu.async_copy(
                table_hbm.at[:, pl.ds(base, _BLK_W)], blk.at[buf], sem
            )

        sems = (sem_b0, sem_b1, sem_b2, sem_b3)

        for k in range(_NBUF):
            fetch_quad(k, k, sems[k])

        pltpu.sync_copy(idx_hbm, idx_all)

        def compact_body(g, count):
            v = idx_all[pl.ds(g * _LANES, _LANES)]
            c = lax.shift_right_logical(v, 7)
            m = (c >= lo) & (c < hi)
            mi = m.astype(jnp.int32)
            pref = plsc.cumsum(mi)
            tgt = jnp.where(m, count + pref - mi, dump_v)
            pk = jnp.bitwise_or(
                lax.shift_left(v - rel0, 14), iota + g * _LANES
            )
            plsc.store_scatter(my_pk, [tgt], pk)
            return count + pref[_LANES - 1]

        n_mine = lax.fori_loop(0, BATCH // _LANES, compact_body, 0)

        for g in range(_CNT_CAP // _LANES):
            counts[pl.ds(g * _LANES, _LANES)] = jnp.zeros((_LANES,), jnp.int32)

        def hist_body(i, carry):
            w = my_pk[pl.ds(i, _LANES)][0]
            c0 = lax.shift_right_logical(w, 21)
            tgt = jnp.where(lane0, _full16(c0), dump_c)
            plsc.addupdate_scatter(counts, [tgt], ones16)
            return carry

        lax.fori_loop(0, n_mine, hist_body, 0)

        carry = jnp.int32(0)
        for g in range(_NBW // _LANES + 1):
            off = g * _LANES
            v = counts[pl.ds(off, _LANES)]
            s = plsc.cumsum(v)
            cursors[pl.ds(off, _LANES)] = s - v + carry
            carry = carry + s[_LANES - 1]

        def place_body(i, carry):
            wv = my_pk[pl.ds(i, _LANES)]
            c0 = lax.shift_right_logical(wv[0], 21)
            c0v = _full16(c0)
            curv = plsc.load_gather(cursors, [c0v])
            tgt = jnp.where(lane0, curv, dump_v)
            plsc.store_scatter(srt_pk, [tgt], _full16(wv[0]))
            tgt2 = jnp.where(lane0, c0v, dump_c)
            plsc.addupdate_scatter(cursors, [tgt2], ones16)
            return carry

        lax.fori_loop(0, n_mine, place_body, 0)

        nquad = lax.div(nblk + _QUAD - 1, _QUAD)

        def drain_one():
            pltpu.make_async_copy(
                ring.at[pl.ds(0, 1), :], out_hbm.at[pl.ds(0, 1), :], sem_row
            ).wait()

        def wait_quad(buf, sem):
            pltpu.make_async_copy(
                table_hbm.at[:, pl.ds(0, _BLK_W)], blk.at[buf], sem
            ).wait()

        def make_site_body(buf, base_rel):
            def site_body(j, carry):
                w = srt_pk[pl.ds(j, _LANES)][0]
                pos = jnp.bitwise_and(w, 16383)
                l = lax.shift_right_logical(w, 14) - base_rel
                slot = jnp.bitwise_and(j, _RING - 1)

                @pl.when(j >= _RING)
                def _():
                    drain_one()

                lv = _full16(l)
                for g in range(EMBEDDING_DIM // _LANES):
                    fv = _iota16() + g * _LANES
                    row = plsc.load_gather(blk.at[buf], [fv, lv])
                    ring[slot, pl.ds(g * _LANES, _LANES)] = row
                pltpu.async_copy(
                    ring.at[pl.ds(slot, 1), :],
                    out_hbm.at[pl.ds(pos, 1), :],
                    sem_row,
                )
                return carry

            return site_body

        def process_quad(p, buf):
            base_rel = quad_base(p) - rel0
            for half in range(_QUAD):
                b = _QUAD * p + half

                @pl.when(b < nblk)
                def _():
                    end = cursors[pl.ds(b, _LANES)][0]
                    cnt = counts[pl.ds(b, _LANES)][0]
                    lax.fori_loop(
                        end - cnt, end, make_site_body(buf, base_rel), 0
                    )

        def quad_loop(q, carry):
            for k in range(_NBUF):
                p = _NBUF * q + k

                @pl.when(p < nquad)
                def _():
                    wait_quad(k, sems[k])
                    process_quad(p, k)

                @pl.when(p + _NBUF < nquad)
                def _():
                    fetch_quad(p + _NBUF, k, sems[k])

            return carry

        lax.fori_loop(0, (nquad + _NBUF - 1) // _NBUF, quad_loop, 0)

        for k in range(_RING):

            @pl.when(n_mine > k)
            def _():
                drain_one()

    return gather_kernel


_gather = jax.jit(_make_kernel())


def kernel(site_ids, embedding_weight):
    return _gather(site_ids.astype(jnp.int32), embedding_weight.T)


# speedup vs baseline: 1.1531x; 1.0512x over previous
import functools

import jax
import jax.numpy as jnp
from jax import lax
from jax.experimental import pallas as pl
from jax.experimental.pallas import tpu as pltpu
from jax.experimental.pallas import tpu_sc as plsc

NUM_SITES = 1000000
EMBEDDING_DIM = 64
BATCH = 16384

_info = plsc.get_sparse_core_info()
_NC, _NS = _info.num_cores, _info.num_subcores
_NW = _NC * _NS
_LANES = 16

_NT = (NUM_SITES + 127) // 128
_NBW = (_NT + _NW - 1) // _NW
_QUAD = 1
_BLK_W = _QUAD * 128
_NBUF = 8

_RING = 16
_CAP = BATCH + _LANES
_CNT_CAP = _NBW + 2 * _LANES


def _iota16():
    return lax.iota(jnp.int32, _LANES)


def _full16(x):
    return jnp.full((_LANES,), x, jnp.int32)


def _make_kernel():
    mesh = plsc.VectorSubcoreMesh(core_axis_name="c", subcore_axis_name="s")

    @functools.partial(
        pl.kernel,
        mesh=mesh,
        out_type=jax.ShapeDtypeStruct((BATCH, EMBEDDING_DIM), jnp.float32),
        scratch_types=[
            pltpu.VMEM((BATCH,), jnp.int32),
            pltpu.VMEM((_CAP + _LANES,), jnp.int32),
            pltpu.VMEM((_CAP + _LANES,), jnp.int32),
            pltpu.VMEM((_CNT_CAP,), jnp.int32),
            pltpu.VMEM((_CNT_CAP,), jnp.int32),
            pltpu.VMEM((_NBUF, EMBEDDING_DIM, _BLK_W), jnp.float32),
            pltpu.VMEM((_RING, EMBEDDING_DIM), jnp.float32),
            pltpu.SemaphoreType.DMA,
            pltpu.SemaphoreType.DMA,
            pltpu.SemaphoreType.DMA,
            pltpu.SemaphoreType.DMA,
            pltpu.SemaphoreType.DMA,
            pltpu.SemaphoreType.DMA,
            pltpu.SemaphoreType.DMA,
            pltpu.SemaphoreType.DMA,
            pltpu.SemaphoreType.DMA,
        ],
        compiler_params=pltpu.CompilerParams(needs_layout_passes=False),
    )
    def gather_kernel(
        idx_hbm,
        table_hbm,
        out_hbm,
        idx_all,
        my_pk,
        srt_pk,
        counts,
        cursors,
        blk,
        ring,
        sem_row,
        sem_b0,
        sem_b1,
        sem_b2,
        sem_b3,
        sem_b4,
        sem_b5,
        sem_b6,
        sem_b7,
    ):
        wid = lax.axis_index("s") * _NC + lax.axis_index("c")
        lo = wid * _NBW
        nblk = jnp.minimum(_NBW, _NT - lo)
        hi = lo + nblk
        rel0 = lo * 128
        iota = _iota16()
        lane0 = iota == 0
        ones16 = _full16(1)
        dump_v = _CAP + iota
        dump_c = (_NBW + _LANES) + iota

        max_base = _NT * 128 - _BLK_W

        def quad_base(p):
            return jnp.minimum((lo + _QUAD * p) * 128, max_base)

        def fetch_quad(p, buf, sem):
            base = pl.multiple_of(quad_base(p), 128)
            pltpu.async_copy(
                table_hbm.at[:, pl.ds(base, _BLK_W)], blk.at[buf], sem
            )

        sems = (sem_b0, sem_b1, sem_b2, sem_b3, sem_b4, sem_b5, sem_b6, sem_b7)

        for k in range(_NBUF):
            fetch_quad(k, k, sems[k])

        pltpu.sync_copy(idx_hbm, idx_all)

        def compact_body(g, count):
            v = idx_all[pl.ds(g * _LANES, _LANES)]
            c = lax.shift_right_logical(v, 7)
            m = (c >= lo) & (c < hi)
            mi = m.astype(jnp.int32)
            pref = plsc.cumsum(mi)
            tgt = jnp.where(m, count + pref - mi, dump_v)
            pk = jnp.bitwise_or(
                lax.shift_left(v - rel0, 14), iota + g * _LANES
            )
            plsc.store_scatter(my_pk, [tgt], pk)
            return count + pref[_LANES - 1]

        n_mine = lax.fori_loop(0, BATCH // _LANES, compact_body, 0)

        for g in range(_CNT_CAP // _LANES):
            counts[pl.ds(g * _LANES, _LANES)] = jnp.zeros((_LANES,), jnp.int32)

        def hist_body(i, carry):
            w = my_pk[pl.ds(i, _LANES)][0]
            c0 = lax.shift_right_logical(w, 21)
            tgt = jnp.where(lane0, _full16(c0), dump_c)
            plsc.addupdate_scatter(counts, [tgt], ones16)
            return carry

        lax.fori_loop(0, n_mine, hist_body, 0)

        carry = jnp.int32(0)
        for g in range(_NBW // _LANES + 1):
            off = g * _LANES
            v = counts[pl.ds(off, _LANES)]
            s = plsc.cumsum(v)
            cursors[pl.ds(off, _LANES)] = s - v + carry
            carry = carry + s[_LANES - 1]

        def place_body(i, carry):
            wv = my_pk[pl.ds(i, _LANES)]
            c0 = lax.shift_right_logical(wv[0], 21)
            c0v = _full16(c0)
            curv = plsc.load_gather(cursors, [c0v])
            tgt = jnp.where(lane0, curv, dump_v)
            plsc.store_scatter(srt_pk, [tgt], _full16(wv[0]))
            tgt2 = jnp.where(lane0, c0v, dump_c)
            plsc.addupdate_scatter(cursors, [tgt2], ones16)
            return carry

        lax.fori_loop(0, n_mine, place_body, 0)

        nquad = lax.div(nblk + _QUAD - 1, _QUAD)

        def drain_one():
            pltpu.make_async_copy(
                ring.at[pl.ds(0, 1), :], out_hbm.at[pl.ds(0, 1), :], sem_row
            ).wait()

        def wait_quad(buf, sem):
            pltpu.make_async_copy(
                table_hbm.at[:, pl.ds(0, _BLK_W)], blk.at[buf], sem
            ).wait()

        def make_site_body(buf, base_rel):
            def site_body(j, carry):
                w = srt_pk[pl.ds(j, _LANES)][0]
                pos = jnp.bitwise_and(w, 16383)
                l = lax.shift_right_logical(w, 14) - base_rel
                slot = jnp.bitwise_and(j, _RING - 1)

                @pl.when(j >= _RING)
                def _():
                    drain_one()

                lv = _full16(l)
                for g in range(EMBEDDING_DIM // _LANES):
                    fv = _iota16() + g * _LANES
                    row = plsc.load_gather(blk.at[buf], [fv, lv])
                    ring[slot, pl.ds(g * _LANES, _LANES)] = row
                pltpu.async_copy(
                    ring.at[pl.ds(slot, 1), :],
                    out_hbm.at[pl.ds(pos, 1), :],
                    sem_row,
                )
                return carry

            return site_body

        def process_quad(p, buf):
            base_rel = quad_base(p) - rel0
            for half in range(_QUAD):
                b = _QUAD * p + half

                @pl.when(b < nblk)
                def _():
                    end = cursors[pl.ds(b, _LANES)][0]
                    cnt = counts[pl.ds(b, _LANES)][0]
                    lax.fori_loop(
                        end - cnt, end, make_site_body(buf, base_rel), 0
                    )

        def quad_loop(q, carry):
            for k in range(_NBUF):
                p = _NBUF * q + k

                @pl.when(p < nquad)
                def _():
                    wait_quad(k, sems[k])
                    process_quad(p, k)

                @pl.when(p + _NBUF < nquad)
                def _():
                    fetch_quad(p + _NBUF, k, sems[k])

            return carry

        lax.fori_loop(0, (nquad + _NBUF - 1) // _NBUF, quad_loop, 0)

        for k in range(_RING):

            @pl.when(n_mine > k)
            def _():
                drain_one()

    return gather_kernel


_gather = jax.jit(_make_kernel())


def kernel(site_ids, embedding_weight):
    return _gather(site_ids.astype(jnp.int32), embedding_weight.T)
